# baseline (speedup 1.0000x reference)
import jax
import jax.numpy as jnp
from jax.experimental import pallas as pl
from jax.experimental.pallas import tpu as pltpu

_N = 4096
_BLK = 512
_ATOL = 1e-8


def _check_kernel(x_ref, out_ref):
    bi = pl.program_id(0)
    row0 = bi * _BLK
    i = row0 + jax.lax.broadcasted_iota(jnp.int32, (_BLK, _N), 0)
    j = jax.lax.broadcasted_iota(jnp.int32, (_BLK, _N), 1)
    mask = i >= j
    r = jnp.where(j > i, x_ref[...], 0.0)
    bad = mask & jnp.logical_not(jnp.abs(r) <= _ATOL)
    anybad = jnp.any(bad).astype(jnp.int32)

    @pl.when(bi == 0)
    def _init():
        out_ref[0, 0] = 1 - anybad

    @pl.when(bi != 0)
    def _acc():
        out_ref[0, 0] = out_ref[0, 0] * (1 - anybad)


def kernel(x):
    ok = pl.pallas_call(
        _check_kernel,
        grid=(_N // _BLK,),
        in_specs=[pl.BlockSpec((_BLK, _N), lambda bi: (bi, 0))],
        out_specs=pl.BlockSpec(
            (1, 1), lambda bi: (0, 0), memory_space=pltpu.SMEM
        ),
        out_shape=jax.ShapeDtypeStruct((1, 1), jnp.int32),
    )(x)
    return jnp.reshape(ok != 0, (1,))


# speedup vs baseline: 3.1231x; 3.1231x over previous
import jax
import jax.numpy as jnp
from jax.experimental import pallas as pl
from jax.experimental.pallas import tpu as pltpu

_N = 4096
_BLK = 512
_ATOL = 1e-8


def _check_kernel(x_ref, out_ref):
    bi = pl.program_id(0)
    i = jax.lax.broadcasted_iota(jnp.int32, (_BLK, _BLK), 0)
    j = jax.lax.broadcasted_iota(jnp.int32, (_BLK, _BLK), 1)
    mask = i >= j
    r = jnp.where(j > i, x_ref[...], 0.0)
    bad = mask & jnp.logical_not(jnp.abs(r) <= _ATOL)
    anybad = jnp.any(bad).astype(jnp.int32)

    @pl.when(bi == 0)
    def _init():
        out_ref[0, 0] = 1 - anybad

    @pl.when(bi != 0)
    def _acc():
        out_ref[0, 0] = out_ref[0, 0] * (1 - anybad)


def kernel(x):
    ok = pl.pallas_call(
        _check_kernel,
        grid=(_N // _BLK,),
        in_specs=[pl.BlockSpec((_BLK, _BLK), lambda bi: (bi, bi))],
        out_specs=pl.BlockSpec(
            (1, 1), lambda bi: (0, 0), memory_space=pltpu.SMEM
        ),
        out_shape=jax.ShapeDtypeStruct((1, 1), jnp.int32),
    )(x)
    return jnp.reshape(ok != 0, (1,))
